# baseline (device time: 51158 ns/iter reference)
import jax
import jax.numpy as jnp
from jax import lax
from jax.experimental import pallas as pl
from jax.experimental.pallas import tpu as pltpu

N_GLOBAL = 4096
EPS = 1e-5
BM = 1024


def kernel(x, gamma):
    m, n = x.shape
    n_tiles = m // BM
    sub = BM // 128

    def body(x_ref, gamma_ref, out_ref, ssq_ref, recv_ref, send_sems, recv_sems):
        k = pl.program_id(0)
        slot = lax.rem(k, 2)
        my_x = lax.axis_index("x")
        my_y = lax.axis_index("y")
        nbr = (my_x, 1 - my_y)

        xv = x_ref[:, :]
        ssq_col = jnp.sum(xv * xv, axis=1, keepdims=True)
        stacked = jnp.concatenate(
            [ssq_col[g * 128:(g + 1) * 128, :] for g in range(sub)], axis=1
        )
        ssq_ref[slot, :, :] = jnp.transpose(stacked)

        @pl.when(k == 0)
        def _():
            barrier_sem = pltpu.get_barrier_semaphore()
            pl.semaphore_signal(
                barrier_sem, inc=1, device_id=nbr,
                device_id_type=pl.DeviceIdType.MESH,
            )
            pl.semaphore_wait(barrier_sem, 1)

        rdma = pltpu.make_async_remote_copy(
            src_ref=ssq_ref.at[slot],
            dst_ref=recv_ref.at[slot],
            send_sem=send_sems.at[slot],
            recv_sem=recv_sems.at[slot],
            device_id=nbr,
            device_id_type=pl.DeviceIdType.MESH,
        )
        rdma.start()
        rdma.wait()

        total8 = ssq_ref[slot, :, :] + recv_ref[slot, :, :]
        invT = lax.rsqrt(
            jnp.transpose(total8) * (1.0 / N_GLOBAL) + EPS
        )
        inv_col = jnp.concatenate(
            [invT[:, g:g + 1] for g in range(sub)], axis=0
        )
        out_ref[:, :] = (xv * gamma_ref[:, :] * inv_col).astype(jnp.bfloat16)

    return pl.pallas_call(
        body,
        grid=(n_tiles,),
        out_shape=jax.ShapeDtypeStruct((m, n), jnp.bfloat16),
        in_specs=[
            pl.BlockSpec((BM, n), lambda k: (k, 0)),
            pl.BlockSpec((1, n), lambda k: (0, 0)),
        ],
        out_specs=pl.BlockSpec((BM, n), lambda k: (k, 0)),
        scratch_shapes=[
            pltpu.VMEM((2, sub, 128), jnp.float32),
            pltpu.VMEM((2, sub, 128), jnp.float32),
            pltpu.SemaphoreType.DMA((2,)),
            pltpu.SemaphoreType.DMA((2,)),
        ],
        compiler_params=pltpu.CompilerParams(
            collective_id=0, vmem_limit_bytes=50 * 1024 * 1024
        ),
    )(x, gamma.reshape(1, n))


# device time: 45888 ns/iter; 1.1148x vs baseline; 1.1148x over previous
import jax
import jax.numpy as jnp
from jax import lax
from jax.experimental import pallas as pl
from jax.experimental.pallas import tpu as pltpu

N_GLOBAL = 4096
EPS = 1e-5
BM = 1536


def kernel(x, gamma):
    m, n = x.shape
    n_tiles = m // BM
    sub = BM // 128

    def body(x_ref, gamma_ref, out_ref, ssq_ref, recv_ref, send_sems, recv_sems):
        k = pl.program_id(0)
        slot = lax.rem(k, 2)
        my_x = lax.axis_index("x")
        my_y = lax.axis_index("y")
        nbr = (my_x, 1 - my_y)

        xv = x_ref[:, :]
        ssq_col = jnp.sum(xv * xv, axis=1, keepdims=True)
        stacked = jnp.concatenate(
            [ssq_col[g * 128:(g + 1) * 128, :] for g in range(sub)], axis=1
        )
        ssq_ref[slot, :, :] = jnp.transpose(stacked)

        @pl.when(k == 0)
        def _():
            barrier_sem = pltpu.get_barrier_semaphore()
            pl.semaphore_signal(
                barrier_sem, inc=1, device_id=nbr,
                device_id_type=pl.DeviceIdType.MESH,
            )
            pl.semaphore_wait(barrier_sem, 1)

        rdma = pltpu.make_async_remote_copy(
            src_ref=ssq_ref.at[slot],
            dst_ref=recv_ref.at[slot],
            send_sem=send_sems.at[slot],
            recv_sem=recv_sems.at[slot],
            device_id=nbr,
            device_id_type=pl.DeviceIdType.MESH,
        )
        rdma.start()

        xg = xv * gamma_ref[:, :]

        rdma.wait_recv()

        total8 = ssq_ref[slot, :, :] + recv_ref[slot, :, :]
        invT = lax.rsqrt(
            jnp.transpose(total8) * (1.0 / N_GLOBAL) + EPS
        )
        inv_col = jnp.concatenate(
            [invT[:, g:g + 1] for g in range(sub)], axis=0
        )
        out_ref[:, :] = (xg * inv_col).astype(jnp.bfloat16)
        rdma.wait_send()

    return pl.pallas_call(
        body,
        grid=(n_tiles,),
        out_shape=jax.ShapeDtypeStruct((m, n), jnp.bfloat16),
        in_specs=[
            pl.BlockSpec((BM, n), lambda k: (k, 0)),
            pl.BlockSpec((1, n), lambda k: (0, 0)),
        ],
        out_specs=pl.BlockSpec((BM, n), lambda k: (k, 0)),
        scratch_shapes=[
            pltpu.VMEM((2, sub, 128), jnp.float32),
            pltpu.VMEM((2, sub, 128), jnp.float32),
            pltpu.SemaphoreType.DMA((2,)),
            pltpu.SemaphoreType.DMA((2,)),
        ],
        compiler_params=pltpu.CompilerParams(
            collective_id=0, vmem_limit_bytes=60 * 1024 * 1024
        ),
    )(x, gamma.reshape(1, n))
